# baseline (device time: 71659 ns/iter reference)
import jax
import jax.numpy as jnp
from jax import lax
from jax.experimental import pallas as pl
from jax.experimental.pallas import tpu as pltpu

N_DEV = 4


def kernel(x, w_mat):
    m_glob, k_shard = x.shape
    k_glob, n = w_mat.shape
    m_blk = m_glob // N_DEV

    def body(x_hbm, w_hbm, out_ref, xdb, xbf, wdb, rbuf,
             xsems, wsems, send_sems, recv_sems):
        my = lax.axis_index("i")

        barrier_sem = pltpu.get_barrier_semaphore()
        for d in range(1, N_DEV):
            pl.semaphore_signal(
                barrier_sem, inc=1,
                device_id=((my + d) % N_DEV,),
                device_id_type=pl.DeviceIdType.MESH,
            )

        blocks = [(my + 1) % N_DEV, (my + 3) % N_DEV, (my + 2) % N_DEV, my]
        dists = [1, 3, 2, None]
        xcopies = [
            pltpu.make_async_copy(
                x_hbm.at[pl.ds(blocks[i] * m_blk, m_blk)],
                xdb.at[i % 2],
                xsems.at[i % 2],
            )
            for i in range(4)
        ]
        xcopies[0].start()
        xcopies[1].start()

        pl.semaphore_wait(barrier_sem, N_DEV - 1)

        rdmas = {}
        for i in range(4):
            xcopies[i].wait()
            xbf[i] = xdb[i % 2].astype(jnp.bfloat16)
            if i + 2 < 4:
                xcopies[i + 2].start()
            if dists[i] is not None:
                d = dists[i]
                rdma = pltpu.make_async_remote_copy(
                    src_ref=xbf.at[i],
                    dst_ref=rbuf.at[d - 1],
                    send_sem=send_sems.at[d - 1],
                    recv_sem=recv_sems.at[d - 1],
                    device_id=(blocks[i],),
                    device_id_type=pl.DeviceIdType.MESH,
                )
                rdma.start()
                rdmas[d] = rdma

        k_order = [my, (my + 3) % N_DEV, (my + 1) % N_DEV, (my + 2) % N_DEV]
        recv_dist = [None, 1, 3, 2]
        wcopies = [
            pltpu.make_async_copy(
                w_hbm.at[pl.ds(k_order[i] * k_shard, k_shard)],
                wdb.at[i % 2],
                wsems.at[i % 2],
            )
            for i in range(4)
        ]
        wcopies[0].start()
        wcopies[1].start()

        for i in range(4):
            wcopies[i].wait()
            wbf = wdb[i % 2].astype(jnp.bfloat16)
            if i + 2 < 4:
                wcopies[i + 2].start()
            if recv_dist[i] is None:
                xsrc = xbf[3]
            else:
                d = recv_dist[i]
                rdmas[d].wait()
                xsrc = rbuf[d - 1]
            contrib = jnp.dot(xsrc, wbf, preferred_element_type=jnp.float32)
            if i == 0:
                out_ref[:, :] = contrib
            else:
                out_ref[:, :] = out_ref[:, :] + contrib

        acc = out_ref[:, :]
        c = 0.7978845608028654
        out_ref[:, :] = 0.5 * acc * (1.0 + jnp.tanh(c * (acc + 0.044715 * acc ** 3)))

    return pl.pallas_call(
        body,
        out_shape=jax.ShapeDtypeStruct((m_blk, n), jnp.float32),
        in_specs=[
            pl.BlockSpec(memory_space=pl.ANY),
            pl.BlockSpec(memory_space=pl.ANY),
        ],
        out_specs=pl.BlockSpec(memory_space=pltpu.VMEM),
        scratch_shapes=[
            pltpu.VMEM((2, m_blk, k_shard), jnp.float32),
            pltpu.VMEM((4, m_blk, k_shard), jnp.bfloat16),
            pltpu.VMEM((2, k_shard, n), jnp.float32),
            pltpu.VMEM((N_DEV - 1, m_blk, k_shard), jnp.bfloat16),
            pltpu.SemaphoreType.DMA((2,)),
            pltpu.SemaphoreType.DMA((2,)),
            pltpu.SemaphoreType.DMA((N_DEV - 1,)),
            pltpu.SemaphoreType.DMA((N_DEV - 1,)),
        ],
        compiler_params=pltpu.CompilerParams(
            collective_id=0,
            vmem_limit_bytes=63 * 1024 * 1024,
        ),
    )(x, w_mat)


# device time: 69449 ns/iter; 1.0318x vs baseline; 1.0318x over previous
import jax
import jax.numpy as jnp
from jax import lax
from jax.experimental import pallas as pl
from jax.experimental.pallas import tpu as pltpu

N_DEV = 4


def kernel(x, w_mat):
    m_glob, k_shard = x.shape
    k_glob, n = w_mat.shape
    m_blk = m_glob // N_DEV
    half = m_blk // 2

    def body(x_hbm, w_hbm, out_ref, xdb, xbf, wdb, rbuf,
             xsems, wsems, send_sems, recv_sems):
        my = lax.axis_index("i")

        barrier_sem = pltpu.get_barrier_semaphore()
        for d in range(1, N_DEV):
            pl.semaphore_signal(
                barrier_sem, inc=1,
                device_id=((my + d) % N_DEV,),
                device_id_type=pl.DeviceIdType.MESH,
            )

        blocks = [(my + 1) % N_DEV, (my + 3) % N_DEV, (my + 2) % N_DEV, my]
        dists = [1, 3, 2, None]

        xcopies = [
            pltpu.make_async_copy(
                x_hbm.at[pl.ds(blocks[0] * m_blk, half)],
                xdb.at[0, pl.ds(0, half)],
                xsems.at[0],
            ),
            pltpu.make_async_copy(
                x_hbm.at[pl.ds(blocks[0] * m_blk + half, half)],
                xdb.at[0, pl.ds(half, half)],
                xsems.at[1],
            ),
        ] + [
            pltpu.make_async_copy(
                x_hbm.at[pl.ds(blocks[i] * m_blk, m_blk)],
                xdb.at[i % 2],
                xsems.at[i + 1],
            )
            for i in range(1, 4)
        ]
        xcopies[0].start()
        xcopies[1].start()
        xcopies[2].start()

        pl.semaphore_wait(barrier_sem, N_DEV - 1)

        rdmas = {}
        for i in range(4):
            for h in range(2):
                if i == 0:
                    xcopies[h].wait()
                elif h == 0:
                    xcopies[i + 1].wait()
                xbf[2 * i + h] = xdb[i % 2, pl.ds(h * half, half)].astype(
                    jnp.bfloat16
                )
                if dists[i] is not None:
                    s = 2 * i + h
                    rdma = pltpu.make_async_remote_copy(
                        src_ref=xbf.at[s],
                        dst_ref=rbuf.at[s],
                        send_sem=send_sems.at[s],
                        recv_sem=recv_sems.at[s],
                        device_id=(blocks[i],),
                        device_id_type=pl.DeviceIdType.MESH,
                    )
                    rdma.start()
                    rdmas[(dists[i], h)] = rdma
            if i + 2 < 4:
                xcopies[i + 3].start()

        k_order = [my, (my + 3) % N_DEV, (my + 1) % N_DEV, (my + 2) % N_DEV]
        recv_dist = [None, 1, 3, 2]
        wcopies = [
            pltpu.make_async_copy(
                w_hbm.at[pl.ds(k_order[i] * k_shard, k_shard)],
                wdb.at[i % 2],
                wsems.at[i % 2],
            )
            for i in range(4)
        ]
        wcopies[0].start()
        wcopies[1].start()

        c = 0.7978845608028654
        for i in range(4):
            wcopies[i].wait()
            wbf = wdb[i % 2].astype(jnp.bfloat16)
            if i + 2 < 4:
                wcopies[i + 2].start()
            for h in range(2):
                if recv_dist[i] is None:
                    xsrc = xbf[6 + h]
                else:
                    d = recv_dist[i]
                    rdmas[(d, h)].wait()
                    xsrc = rbuf[2 * (i - 1) + h]
                contrib = jnp.dot(
                    xsrc, wbf, preferred_element_type=jnp.float32
                )
                rows = pl.ds(h * half, half)
                if i == 0:
                    out_ref[rows, :] = contrib
                else:
                    out_ref[rows, :] = out_ref[rows, :] + contrib
                if i == 3:
                    y = out_ref[rows, :]
                    out_ref[rows, :] = (
                        0.5 * y * (1.0 + jnp.tanh(c * (y + 0.044715 * y ** 3)))
                    )

    return pl.pallas_call(
        body,
        out_shape=jax.ShapeDtypeStruct((m_blk, n), jnp.float32),
        in_specs=[
            pl.BlockSpec(memory_space=pl.ANY),
            pl.BlockSpec(memory_space=pl.ANY),
        ],
        out_specs=pl.BlockSpec(memory_space=pltpu.VMEM),
        scratch_shapes=[
            pltpu.VMEM((2, m_blk, k_shard), jnp.float32),
            pltpu.VMEM((8, half, k_shard), jnp.bfloat16),
            pltpu.VMEM((2, k_shard, n), jnp.float32),
            pltpu.VMEM((6, half, k_shard), jnp.bfloat16),
            pltpu.SemaphoreType.DMA((5,)),
            pltpu.SemaphoreType.DMA((2,)),
            pltpu.SemaphoreType.DMA((6,)),
            pltpu.SemaphoreType.DMA((6,)),
        ],
        compiler_params=pltpu.CompilerParams(
            collective_id=0,
            vmem_limit_bytes=63 * 1024 * 1024,
        ),
    )(x, w_mat)


# device time: 67938 ns/iter; 1.0548x vs baseline; 1.0222x over previous
import jax
import jax.numpy as jnp
from jax import lax
from jax.experimental import pallas as pl
from jax.experimental.pallas import tpu as pltpu

N_DEV = 4
S = 4


def kernel(x, w_mat):
    m_glob, k_shard = x.shape
    k_glob, n = w_mat.shape
    m_blk = m_glob // N_DEV
    sub = m_blk // S

    def body(x_hbm, w_hbm, out_ref, xdb, xbf, wdb, rbuf,
             xsems, wsems, send_sems, recv_sems):
        my = lax.axis_index("i")

        barrier_sem = pltpu.get_barrier_semaphore()
        for d in range(1, N_DEV):
            pl.semaphore_signal(
                barrier_sem, inc=1,
                device_id=((my + d) % N_DEV,),
                device_id_type=pl.DeviceIdType.MESH,
            )

        blocks = [(my + 1) % N_DEV, (my + 3) % N_DEV, (my + 2) % N_DEV, my]
        dists = [1, 3, 2, None]

        xcopies = [
            pltpu.make_async_copy(
                x_hbm.at[pl.ds(blocks[0] * m_blk, sub)],
                xdb.at[0, pl.ds(0, sub)],
                xsems.at[0],
            ),
            pltpu.make_async_copy(
                x_hbm.at[pl.ds(blocks[0] * m_blk + sub, m_blk - sub)],
                xdb.at[0, pl.ds(sub, m_blk - sub)],
                xsems.at[1],
            ),
        ] + [
            pltpu.make_async_copy(
                x_hbm.at[pl.ds(blocks[i] * m_blk, m_blk)],
                xdb.at[i % 2],
                xsems.at[i + 1],
            )
            for i in range(1, 4)
        ]
        xcopies[0].start()
        xcopies[1].start()
        xcopies[2].start()

        pl.semaphore_wait(barrier_sem, N_DEV - 1)

        rdmas = {}
        for i in range(4):
            for q in range(S):
                if i == 0:
                    if q == 0:
                        xcopies[0].wait()
                    elif q == 1:
                        xcopies[1].wait()
                elif q == 0:
                    xcopies[i + 1].wait()
                s = S * i + q
                xbf[s] = xdb[i % 2, pl.ds(q * sub, sub)].astype(jnp.bfloat16)
                if dists[i] is not None:
                    rdma = pltpu.make_async_remote_copy(
                        src_ref=xbf.at[s],
                        dst_ref=rbuf.at[s],
                        send_sem=send_sems.at[s],
                        recv_sem=recv_sems.at[s],
                        device_id=(blocks[i],),
                        device_id_type=pl.DeviceIdType.MESH,
                    )
                    rdma.start()
                    rdmas[(dists[i], q)] = rdma
            if i + 2 < 4:
                xcopies[i + 3].start()

        k_order = [my, (my + 3) % N_DEV, (my + 1) % N_DEV, (my + 2) % N_DEV]
        recv_dist = [None, 1, 3, 2]
        wcopies = [
            pltpu.make_async_copy(
                w_hbm.at[pl.ds(k_order[i] * k_shard, k_shard)],
                wdb.at[i % 2],
                wsems.at[i % 2],
            )
            for i in range(4)
        ]
        wcopies[0].start()
        wcopies[1].start()

        c = 0.7978845608028654
        for i in range(4):
            wcopies[i].wait()
            wbf = wdb[i % 2].astype(jnp.bfloat16)
            if i + 2 < 4:
                wcopies[i + 2].start()
            for q in range(S):
                if recv_dist[i] is None:
                    xsrc = xbf[3 * S + q]
                else:
                    d = recv_dist[i]
                    rdmas[(d, q)].wait()
                    xsrc = rbuf[S * (i - 1) + q]
                contrib = jnp.dot(
                    xsrc, wbf, preferred_element_type=jnp.float32
                )
                rows = pl.ds(q * sub, sub)
                if i == 0:
                    out_ref[rows, :] = contrib
                else:
                    out_ref[rows, :] = out_ref[rows, :] + contrib
                if i == 3:
                    y = out_ref[rows, :]
                    out_ref[rows, :] = (
                        0.5 * y * (1.0 + jnp.tanh(c * (y + 0.044715 * y ** 3)))
                    )

    return pl.pallas_call(
        body,
        out_shape=jax.ShapeDtypeStruct((m_blk, n), jnp.float32),
        in_specs=[
            pl.BlockSpec(memory_space=pl.ANY),
            pl.BlockSpec(memory_space=pl.ANY),
        ],
        out_specs=pl.BlockSpec(memory_space=pltpu.VMEM),
        scratch_shapes=[
            pltpu.VMEM((2, m_blk, k_shard), jnp.float32),
            pltpu.VMEM((4 * S, sub, k_shard), jnp.bfloat16),
            pltpu.VMEM((2, k_shard, n), jnp.float32),
            pltpu.VMEM((3 * S, sub, k_shard), jnp.bfloat16),
            pltpu.SemaphoreType.DMA((5,)),
            pltpu.SemaphoreType.DMA((2,)),
            pltpu.SemaphoreType.DMA((3 * S,)),
            pltpu.SemaphoreType.DMA((3 * S,)),
        ],
        compiler_params=pltpu.CompilerParams(
            collective_id=0,
            vmem_limit_bytes=63 * 1024 * 1024,
        ),
    )(x, w_mat)
